# baseline (speedup 1.0000x reference)
import functools

import jax
import jax.numpy as jnp
from jax.experimental import pallas as pl

_NEG_INF = float("-inf")
_BIG = 1 << 30
_SUB = 8


def _dlr_body(xt_ref, y_ref, o_ref):
    xt = xt_ref[...]
    yv = y_ref[...][None, :]
    C, B = xt.shape
    n_chunks = C // _SUB

    subl = jax.lax.broadcasted_iota(jnp.int32, (_SUB, B), 0)
    neg = jnp.full((_SUB, B), _NEG_INF, dtype=jnp.float32)

    ch = xt[:_SUB, :]
    m0, m1, m2 = ch, neg, neg
    cacc = jnp.where(subl == yv, ch, _NEG_INF)

    for k in range(1, n_chunks):
        ch = xt[k * _SUB:(k + 1) * _SUB, :]
        cacc = jnp.maximum(cacc, jnp.where(subl == yv - (k * _SUB), ch, _NEG_INF))
        t1 = jnp.minimum(m0, ch)
        m0 = jnp.maximum(m0, ch)
        t2 = jnp.minimum(m1, t1)
        m1 = jnp.maximum(m1, t1)
        m2 = jnp.maximum(m2, t2)

    c = jnp.max(cacc, axis=0, keepdims=True)

    u = jnp.concatenate([m0, m1, m2], axis=0)
    urow = jax.lax.broadcasted_iota(jnp.int32, u.shape, 0)
    p0 = jnp.max(u, axis=0, keepdims=True)
    a0 = jnp.min(jnp.where(u == p0, urow, _BIG), axis=0, keepdims=True)
    u1 = jnp.where(urow == a0, _NEG_INF, u)
    p1 = jnp.max(u1, axis=0, keepdims=True)
    a1 = jnp.min(jnp.where(u1 == p1, urow, _BIG), axis=0, keepdims=True)
    u2 = jnp.where(urow == a1, _NEG_INF, u1)
    p2 = jnp.max(u2, axis=0, keepdims=True)

    target = jnp.where(c == p0, p1, p0)
    o_ref[...] = ((target - c) / (p0 - p2))[0]


@functools.partial(jax.jit, static_argnames=("block_cols",))
def _dlr_tc(prediction, y, block_cols=4096):
    n, c = prediction.shape
    xt = prediction.T
    return pl.pallas_call(
        _dlr_body,
        grid=(n // block_cols,),
        in_specs=[
            pl.BlockSpec((c, block_cols), lambda i: (0, i)),
            pl.BlockSpec((block_cols,), lambda i: (i,)),
        ],
        out_specs=pl.BlockSpec((block_cols,), lambda i: (i,)),
        out_shape=jax.ShapeDtypeStruct((n,), jnp.float32),
    )(xt, y)


def kernel(prediction, y):
    return _dlr_tc(prediction, y)


# speedup vs baseline: 1.0931x; 1.0931x over previous
import functools

import jax
import jax.numpy as jnp
from jax.experimental import pallas as pl

_NEG_INF = float("-inf")
_BIG = 1 << 30
_SUB = 8


def _dlr_body(xt_ref, y_ref, o_ref):
    xt = xt_ref[...]
    yv = y_ref[...][None, :]
    C, B = xt.shape
    n_chunks = C // _SUB

    subl = jax.lax.broadcasted_iota(jnp.int32, (_SUB, B), 0)
    neg = jnp.full((_SUB, B), _NEG_INF, dtype=jnp.float32)

    ch = xt[:_SUB, :]
    m0, m1, m2 = ch, neg, neg
    cacc = jnp.where(subl == yv, ch, _NEG_INF)

    for k in range(1, n_chunks):
        ch = xt[k * _SUB:(k + 1) * _SUB, :]
        cacc = jnp.where(subl == yv - (k * _SUB), ch, cacc)
        t1 = jnp.minimum(m0, ch)
        m0 = jnp.maximum(m0, ch)
        t2 = jnp.minimum(m1, t1)
        m1 = jnp.maximum(m1, t1)
        m2 = jnp.maximum(m2, t2)

    c = jnp.max(cacc, axis=0, keepdims=True)

    u = jnp.concatenate([m0, m1, m2], axis=0)
    urow = jax.lax.broadcasted_iota(jnp.int32, u.shape, 0)
    p0 = jnp.max(u, axis=0, keepdims=True)
    a0 = jnp.min(jnp.where(u == p0, urow, _BIG), axis=0, keepdims=True)
    u1 = jnp.where(urow == a0, _NEG_INF, u)
    p1 = jnp.max(u1, axis=0, keepdims=True)
    a1 = jnp.min(jnp.where(u1 == p1, urow, _BIG), axis=0, keepdims=True)
    u2 = jnp.where(urow == a1, _NEG_INF, u1)
    p2 = jnp.max(u2, axis=0, keepdims=True)

    target = jnp.where(c == p0, p1, p0)
    o_ref[...] = ((target - c) / (p0 - p2))[0]


@functools.partial(jax.jit, static_argnames=("block_cols",))
def _dlr_tc(prediction, y, block_cols=2048):
    n, c = prediction.shape
    xt = prediction.T
    return pl.pallas_call(
        _dlr_body,
        grid=(n // block_cols,),
        in_specs=[
            pl.BlockSpec((c, block_cols), lambda i: (0, i)),
            pl.BlockSpec((block_cols,), lambda i: (i,)),
        ],
        out_specs=pl.BlockSpec((block_cols,), lambda i: (i,)),
        out_shape=jax.ShapeDtypeStruct((n,), jnp.float32),
    )(xt, y)


def kernel(prediction, y):
    return _dlr_tc(prediction, y)
